# baseline (speedup 1.0000x reference)
import functools

import jax
import jax.numpy as jnp
import numpy as np
from jax.experimental import pallas as pl
from jax.experimental.pallas import tpu as pltpu

_S_BIT = 8
_LOG4_INV = float(1.0 / np.log(4.0))
_CHUNK = 32
_HIGH = jax.lax.Precision.HIGHEST


def _vq_body(x_ref, e0_ref, e1_ref, a_ref, b_ref, nn_ref, bd_ref, p_ref,
             pt_ref, o_ref, loss_ref, xne_ref, xno_ref, sq_ref,
             *, n_codes, n_pairs):
    lanes = 128
    x = x_ref[...]

    y = jax.lax.dot_general(
        x, p_ref[...], (((1,), (0,)), ((), ())), precision=_HIGH,
        preferred_element_type=jnp.float32)
    xe = y[:, :lanes]
    xo = y[:, lanes:]

    s2 = xe * xe + xo * xo
    scale2 = jax.lax.dot_general(
        s2, bd_ref[...], (((1,), (0,)), ((), ())), precision=_HIGH,
        preferred_element_type=jnp.float32)
    scale = jnp.sqrt(scale2)
    xne = xe / scale
    xno = xo / scale
    xne_ref[...] = xne
    xno_ref[...] = xno
    xnorm_sum = jnp.sum(xne * xne + xno * xno)

    xb = jnp.clip(scale + 1.0, 1.0, 16.0)
    yb = jnp.log(xb) * np.float32(_LOG4_INV)
    acc = jnp.zeros_like(yb)
    for i_bit in range(_S_BIT):
        t = jnp.floor(yb * np.float32(2.0 ** i_bit))
        bit = t - 2.0 * jnp.floor(t * 0.5)
        acc = acc + bit * np.float32(2.0 ** (-i_bit))
    sq_ref[...] = jnp.exp2(2.0 * acc) - 1.0

    rows = xe.shape[0]
    n_chunks = rows // _CHUNK

    def chunk_body(c, dacc):
        r = pl.ds(c * _CHUNK, _CHUNK)
        xnec = xne_ref[r, :]
        xnoc = xno_ref[r, :]
        bq0 = jnp.full_like(xnec, e0_ref[0])
        bq1 = jnp.full_like(xnec, e1_ref[0])
        bestd = (xnec * a_ref[0] + xnoc * b_ref[0]) + nn_ref[0]
        for j in range(1, n_codes):
            d = (xnec * a_ref[j] + xnoc * b_ref[j]) + nn_ref[j]
            m = d < bestd
            bestd = jnp.minimum(d, bestd)
            bq0 = jnp.where(m, e0_ref[j], bq0)
            bq1 = jnp.where(m, e1_ref[j], bq1)
        sqc = sq_ref[r, :]
        oec = bq0 * sqc
        ooc = bq1 * sqc
        ocat = jnp.concatenate([oec, ooc], axis=1)
        o_ref[r, :] = jax.lax.dot_general(
            ocat, pt_ref[...], (((1,), (0,)), ((), ())), precision=_HIGH,
            preferred_element_type=jnp.float32)
        return dacc + bestd

    dacc = jax.lax.fori_loop(
        0, n_chunks, chunk_body,
        jnp.zeros((_CHUNK, lanes), jnp.float32))

    loss_ref[0] = (jnp.sum(dacc) + xnorm_sum) * np.float32(0.5 / n_pairs)


def kernel(feedback, embed):
    b, p, f = feedback.shape
    n = b * p
    group = f // 2
    n_codes = embed.shape[0]
    lanes = 128
    rows = n * group // lanes

    x256 = feedback.reshape(rows, 2 * lanes)
    e0 = embed[:, 0]
    e1 = embed[:, 1]
    ca = -2.0 * e0
    cb = -2.0 * e1
    cn = e0 * e0 + e1 * e1

    li = jax.lax.broadcasted_iota(jnp.int32, (lanes, lanes), 0)
    mi = jax.lax.broadcasted_iota(jnp.int32, (lanes, lanes), 1)
    bd_ones = (li // group == mi // group).astype(jnp.float32)

    ri = jax.lax.broadcasted_iota(jnp.int32, (2 * lanes, 2 * lanes), 0)
    ci = jax.lax.broadcasted_iota(jnp.int32, (2 * lanes, 2 * lanes), 1)
    lp = ci % lanes
    src = f * (lp // group) + 2 * (lp % group) + (ci // lanes)
    perm = (ri == src).astype(jnp.float32)
    perm_t = perm.T

    in_specs = [
        pl.BlockSpec(memory_space=pltpu.VMEM),
        pl.BlockSpec(memory_space=pltpu.SMEM),
        pl.BlockSpec(memory_space=pltpu.SMEM),
        pl.BlockSpec(memory_space=pltpu.SMEM),
        pl.BlockSpec(memory_space=pltpu.SMEM),
        pl.BlockSpec(memory_space=pltpu.SMEM),
        pl.BlockSpec(memory_space=pltpu.VMEM),
        pl.BlockSpec(memory_space=pltpu.VMEM),
        pl.BlockSpec(memory_space=pltpu.VMEM),
    ]
    out_specs = [
        pl.BlockSpec(memory_space=pltpu.VMEM),
        pl.BlockSpec(memory_space=pltpu.SMEM),
    ]
    o256, loss = pl.pallas_call(
        functools.partial(_vq_body, n_codes=n_codes, n_pairs=rows * lanes),
        in_specs=in_specs,
        out_specs=out_specs,
        out_shape=[
            jax.ShapeDtypeStruct((rows, 2 * lanes), jnp.float32),
            jax.ShapeDtypeStruct((1,), jnp.float32),
        ],
        scratch_shapes=[
            pltpu.VMEM((rows, lanes), jnp.float32),
            pltpu.VMEM((rows, lanes), jnp.float32),
            pltpu.VMEM((rows, lanes), jnp.float32),
        ],
    )(x256, e0, e1, ca, cb, cn, bd_ones, perm, perm_t)

    out = o256.reshape(b, p, f)
    loss = loss.reshape(())
    return (out, loss, loss)


# speedup vs baseline: 1.1067x; 1.1067x over previous
import functools

import jax
import jax.numpy as jnp
import numpy as np
from jax.experimental import pallas as pl
from jax.experimental.pallas import tpu as pltpu

_S_BIT = 8
_LOG4_INV = float(1.0 / np.log(4.0))
_CHUNK = 64
_HIGH = jax.lax.Precision.HIGHEST


def _vq_body(x_ref, e0_ref, e1_ref, a_ref, b_ref, nn_ref, bd_ref, p_ref,
             pt_ref, o_ref, loss_ref, xne_ref, xno_ref, sq_ref,
             *, n_codes, n_pairs):
    lanes = 128
    x = x_ref[...]

    y = jax.lax.dot_general(
        x, p_ref[...], (((1,), (0,)), ((), ())), precision=_HIGH,
        preferred_element_type=jnp.float32)
    xe = y[:, :lanes]
    xo = y[:, lanes:]

    s2 = xe * xe + xo * xo
    scale2 = jax.lax.dot_general(
        s2, bd_ref[...], (((1,), (0,)), ((), ())), precision=_HIGH,
        preferred_element_type=jnp.float32)
    scale = jnp.sqrt(scale2)
    xne = xe / scale
    xno = xo / scale
    xne_ref[...] = xne
    xno_ref[...] = xno
    xnorm_sum = jnp.sum(xne * xne + xno * xno)

    xb = jnp.clip(scale + 1.0, 1.0, 16.0)
    yb = jnp.log(xb) * np.float32(_LOG4_INV)
    acc = jnp.zeros_like(yb)
    for i_bit in range(_S_BIT):
        t = jnp.floor(yb * np.float32(2.0 ** i_bit))
        bit = t - 2.0 * jnp.floor(t * 0.5)
        acc = acc + bit * np.float32(2.0 ** (-i_bit))
    sq_ref[...] = jnp.exp2(2.0 * acc) - 1.0

    rows = xe.shape[0]
    n_chunks = rows // _CHUNK

    def chunk_body(c, dacc):
        r = pl.ds(c * _CHUNK, _CHUNK)
        xnec = xne_ref[r, :]
        xnoc = xno_ref[r, :]
        bq0 = jnp.full_like(xnec, e0_ref[0])
        bq1 = jnp.full_like(xnec, e1_ref[0])
        bestd = (xnec * a_ref[0] + xnoc * b_ref[0]) + nn_ref[0]
        for j in range(1, n_codes):
            d = (xnec * a_ref[j] + xnoc * b_ref[j]) + nn_ref[j]
            m = d < bestd
            bestd = jnp.minimum(d, bestd)
            bq0 = jnp.where(m, e0_ref[j], bq0)
            bq1 = jnp.where(m, e1_ref[j], bq1)
        sqc = sq_ref[r, :]
        oec = bq0 * sqc
        ooc = bq1 * sqc
        ocat = jnp.concatenate([oec, ooc], axis=1)
        o_ref[r, :] = jax.lax.dot_general(
            ocat, pt_ref[...], (((1,), (0,)), ((), ())), precision=_HIGH,
            preferred_element_type=jnp.float32)
        return dacc + bestd

    dacc = jax.lax.fori_loop(
        0, n_chunks, chunk_body,
        jnp.zeros((_CHUNK, lanes), jnp.float32))

    loss_ref[0] = (jnp.sum(dacc) + xnorm_sum) * np.float32(0.5 / n_pairs)


def kernel(feedback, embed):
    b, p, f = feedback.shape
    n = b * p
    group = f // 2
    n_codes = embed.shape[0]
    lanes = 128
    rows = n * group // lanes

    x256 = feedback.reshape(rows, 2 * lanes)
    e0 = embed[:, 0]
    e1 = embed[:, 1]
    ca = -2.0 * e0
    cb = -2.0 * e1
    cn = e0 * e0 + e1 * e1

    li = jax.lax.broadcasted_iota(jnp.int32, (lanes, lanes), 0)
    mi = jax.lax.broadcasted_iota(jnp.int32, (lanes, lanes), 1)
    bd_ones = (li // group == mi // group).astype(jnp.float32)

    ri = jax.lax.broadcasted_iota(jnp.int32, (2 * lanes, 2 * lanes), 0)
    ci = jax.lax.broadcasted_iota(jnp.int32, (2 * lanes, 2 * lanes), 1)
    lp = ci % lanes
    src = f * (lp // group) + 2 * (lp % group) + (ci // lanes)
    perm = (ri == src).astype(jnp.float32)
    perm_t = perm.T

    in_specs = [
        pl.BlockSpec(memory_space=pltpu.VMEM),
        pl.BlockSpec(memory_space=pltpu.SMEM),
        pl.BlockSpec(memory_space=pltpu.SMEM),
        pl.BlockSpec(memory_space=pltpu.SMEM),
        pl.BlockSpec(memory_space=pltpu.SMEM),
        pl.BlockSpec(memory_space=pltpu.SMEM),
        pl.BlockSpec(memory_space=pltpu.VMEM),
        pl.BlockSpec(memory_space=pltpu.VMEM),
        pl.BlockSpec(memory_space=pltpu.VMEM),
    ]
    out_specs = [
        pl.BlockSpec(memory_space=pltpu.VMEM),
        pl.BlockSpec(memory_space=pltpu.SMEM),
    ]
    o256, loss = pl.pallas_call(
        functools.partial(_vq_body, n_codes=n_codes, n_pairs=rows * lanes),
        in_specs=in_specs,
        out_specs=out_specs,
        out_shape=[
            jax.ShapeDtypeStruct((rows, 2 * lanes), jnp.float32),
            jax.ShapeDtypeStruct((1,), jnp.float32),
        ],
        scratch_shapes=[
            pltpu.VMEM((rows, lanes), jnp.float32),
            pltpu.VMEM((rows, lanes), jnp.float32),
            pltpu.VMEM((rows, lanes), jnp.float32),
        ],
    )(x256, e0, e1, ca, cb, cn, bd_ones, perm, perm_t)

    out = o256.reshape(b, p, f)
    loss = loss.reshape(())
    return (out, loss, loss)
